# baseline (speedup 1.0000x reference)
import jax
import jax.numpy as jnp
from jax import lax
from jax.experimental import pallas as pl
from jax.experimental.pallas import tpu as pltpu

D_M = 768
R_CHUNK = 16
N_FULL = 48
NBUF = 2


def _probe_body(we_hbm, wo_hbm, out_ref, webuf, wobuf, sems):

    def start(slot, chunk):
        pltpu.make_async_copy(
            we_hbm.at[pl.ds(chunk * R_CHUNK, R_CHUNK)],
            webuf.at[slot], sems.at[0, slot]).start()
        pltpu.make_async_copy(
            wo_hbm.at[pl.ds(chunk * R_CHUNK, R_CHUNK)],
            wobuf.at[slot], sems.at[1, slot]).start()

    for k in range(NBUF - 1):
        start(k, k)
    out_ref[...] = jnp.zeros_like(out_ref)

    def step(i, carry):
        slot = lax.rem(i, NBUF)
        pltpu.make_async_copy(
            we_hbm.at[pl.ds(0, R_CHUNK)], webuf.at[slot],
            sems.at[0, slot]).wait()
        pltpu.make_async_copy(
            wo_hbm.at[pl.ds(0, R_CHUNK)], wobuf.at[slot],
            sems.at[1, slot]).wait()
        nxt = i + NBUF - 1

        @pl.when(nxt < N_FULL)
        def _():
            start(lax.rem(nxt, NBUF), nxt)

        out_ref[:16] += webuf[slot, :, :D_M] + wobuf[slot, :, :D_M]
        return carry

    lax.fori_loop(0, N_FULL, step, 0)


def _probe(w_e, w_o):
    return pl.pallas_call(
        _probe_body,
        in_specs=[
            pl.BlockSpec(memory_space=pltpu.MemorySpace.HBM),
            pl.BlockSpec(memory_space=pltpu.MemorySpace.HBM),
        ],
        out_specs=pl.BlockSpec((D_M, D_M), lambda: (0, 0)),
        out_shape=jax.ShapeDtypeStruct((D_M, D_M), jnp.float32),
        scratch_shapes=[
            pltpu.VMEM((NBUF, R_CHUNK, 100000), jnp.float32),
            pltpu.VMEM((NBUF, R_CHUNK, 100000), jnp.float32),
            pltpu.SemaphoreType.DMA((2, NBUF)),
        ],
    )(w_e, w_o)


def kernel(x, w_e, w_o, b_o):
    t = _probe(w_e, w_o)
    return jnp.broadcast_to(t[:16, :768].reshape(1, 16, 768), (16, 16, 768))


# speedup vs baseline: 1.0226x; 1.0226x over previous
import functools

import jax
import jax.numpy as jnp
from jax import lax
from jax.experimental import pallas as pl
from jax.experimental.pallas import tpu as pltpu
from jax.experimental.pallas import tpu_sc as plsc

D_M = 768
V_TOT = 100000
V_CHUNK = 2048
N_CHUNK = (V_TOT + V_CHUNK - 1) // V_CHUNK
V_LAST = V_TOT - (N_CHUNK - 1) * V_CHUNK
B_TOK = 256

_DIMS = (((1,), (1,)), ((), ()))


def _mm_body(we_ref, wo_ref, b_ref, out_ref):
    i = pl.program_id(0)

    @pl.when(i == 0)
    def _init():
        out_ref[...] = jnp.broadcast_to(b_ref[...], (D_M, D_M))

    col = lax.broadcasted_iota(jnp.int32, (D_M, V_CHUNK), 1)
    keep = (col < V_LAST) | (i < N_CHUNK - 1)
    we = jnp.where(keep, we_ref[...], 0.0).astype(jnp.bfloat16)
    wo = jnp.where(keep, wo_ref[...], 0.0).astype(jnp.bfloat16)
    out_ref[...] += lax.dot_general(
        we, wo, _DIMS, preferred_element_type=jnp.float32)


def _fused_table(w_e, w_o, b_row):
    return pl.pallas_call(
        _mm_body,
        grid=(N_CHUNK,),
        in_specs=[
            pl.BlockSpec((D_M, V_CHUNK), lambda i: (0, i)),
            pl.BlockSpec((D_M, V_CHUNK), lambda i: (0, i)),
            pl.BlockSpec((1, D_M), lambda i: (0, 0)),
        ],
        out_specs=pl.BlockSpec((D_M, D_M), lambda i: (0, 0)),
        out_shape=jax.ShapeDtypeStruct((D_M, D_M), jnp.float32),
        compiler_params=pltpu.CompilerParams(
            dimension_semantics=("arbitrary",),
        ),
    )(w_e, w_o, b_row)


def _make_sc_gather():
    info = plsc.get_sparse_core_info()
    nc, ns = info.num_cores, info.num_subcores
    nw = nc * ns
    b_per_w = B_TOK // nw
    mesh = plsc.VectorSubcoreMesh(core_axis_name="c", subcore_axis_name="s")

    @functools.partial(
        pl.kernel,
        mesh=mesh,
        out_type=jax.ShapeDtypeStruct((B_TOK, D_M), jnp.float32),
        scratch_types=[
            pltpu.VMEM((b_per_w,), jnp.int32),
            pltpu.VMEM((b_per_w, D_M), jnp.float32),
            pltpu.SemaphoreType.DMA,
        ],
    )
    def gather_k(table_hbm, idx_hbm, out_hbm, idx_v, rows_v, sem):
        wid = lax.axis_index("s") * nc + lax.axis_index("c")
        base = wid * b_per_w
        pltpu.sync_copy(idx_hbm.at[pl.ds(base, b_per_w)], idx_v)
        pltpu.async_copy(table_hbm.at[idx_v], rows_v, sem).wait()
        pltpu.sync_copy(rows_v, out_hbm.at[pl.ds(base, b_per_w)])

    return gather_k


_sc_gather = None


def kernel(x, w_e, w_o, b_o):
    global _sc_gather
    if _sc_gather is None:
        _sc_gather = _make_sc_gather()
    table = _fused_table(w_e, w_o, b_o.reshape(1, D_M))
    idx = x.reshape(-1).astype(jnp.int32)
    out = _sc_gather(table, idx)
    return out.reshape(x.shape[0], x.shape[1], D_M)
